# Initial kernel scaffold; baseline (speedup 1.0000x reference)
import jax
import jax.numpy as jnp
from jax.experimental import pallas as pl


def kernel(geo_x, euc_x, edge_index, Wq, bq, Wk, bk, Wv, bv, Ws, bs, ln_w, ln_b):
    raise NotImplementedError("write your pallas kernel here")


# speedup vs baseline: 1.6198x; 1.6198x over previous
import functools

import jax
import jax.numpy as jnp
from jax import lax
from jax.experimental import pallas as pl
from jax.experimental.pallas import tpu as pltpu

N_NODES = 10000
N_EDGES = 160000
DIM = 256


_ROWS = 1000


def _mm_body(eu_ref, geo_ref, wqs_ref, wkv_ref, bqs_ref, bkv_ref,
             q_ref, skip_ref, k_ref, vab_ref):
    qs = jnp.dot(eu_ref[...], wqs_ref[...],
                 preferred_element_type=jnp.float32) + bqs_ref[...]
    kv = jnp.dot(geo_ref[...], wkv_ref[...],
                 preferred_element_type=jnp.float32) + bkv_ref[...]
    q_ref[...] = qs[:, :DIM]
    skip_ref[...] = qs[:, DIM:]
    k_ref[...] = kv[:, :DIM]
    vab_ref[0] = kv[:, DIM:DIM + 128]
    vab_ref[1] = kv[:, DIM + 128:]


def _projections(euc_x, geo_x, Wq, bq, Wk, bk, Wv, bv, Ws, bs):
    wqs = jnp.concatenate([Wq, Ws], axis=1)
    wkv = jnp.concatenate([Wk, Wv], axis=1)
    bqs = jnp.concatenate([bq, bs])[None, :]
    bkv = jnp.concatenate([bk, bv])[None, :]
    grid = N_NODES // _ROWS
    q, skip, k, vab = pl.pallas_call(
        _mm_body,
        grid=(grid,),
        in_specs=[
            pl.BlockSpec((_ROWS, DIM), lambda i: (i, 0)),
            pl.BlockSpec((_ROWS, DIM), lambda i: (i, 0)),
            pl.BlockSpec((DIM, 2 * DIM), lambda i: (0, 0)),
            pl.BlockSpec((DIM, 2 * DIM), lambda i: (0, 0)),
            pl.BlockSpec((1, 2 * DIM), lambda i: (0, 0)),
            pl.BlockSpec((1, 2 * DIM), lambda i: (0, 0)),
        ],
        out_specs=[
            pl.BlockSpec((_ROWS, DIM), lambda i: (i, 0)),
            pl.BlockSpec((_ROWS, DIM), lambda i: (i, 0)),
            pl.BlockSpec((_ROWS, DIM), lambda i: (i, 0)),
            pl.BlockSpec((2, _ROWS, 128), lambda i: (0, i, 0)),
        ],
        out_shape=[
            jax.ShapeDtypeStruct((N_NODES, DIM), jnp.float32),
            jax.ShapeDtypeStruct((N_NODES, DIM), jnp.float32),
            jax.ShapeDtypeStruct((N_NODES, DIM), jnp.float32),
            jax.ShapeDtypeStruct((2, N_NODES, 128), jnp.float32),
        ],
    )(euc_x, geo_x, wqs, wkv, bqs, bkv)
    return q, skip, k, vab


def _finalize_body(acc_ref, den_ref, skip_ref, lnw_ref, lnb_ref, out_ref):
    den = jnp.sum(den_ref[...], axis=0).reshape(N_NODES, 1) + 1e-16
    pre = jnp.concatenate([acc_ref[0], acc_ref[1]], axis=1) / den
    pre = pre + skip_ref[...]
    mu = jnp.mean(pre)
    cen = pre - mu
    sig = jnp.sqrt(jnp.mean(cen * cen))
    out = cen / (sig + 1e-5) * lnw_ref[...] + lnb_ref[...]
    out_ref[...] = jnp.maximum(out, 0.0)


def _finalize(acc, den_parts, skip, ln_w, ln_b):
    return pl.pallas_call(
        _finalize_body,
        out_shape=jax.ShapeDtypeStruct((N_NODES, DIM), jnp.float32),
    )(acc, den_parts, skip, ln_w[None, :], ln_b[None, :])


def _sparse_middle(q, k, vab, src, dst):
    v = jnp.concatenate([vab[0], vab[1]], axis=1)
    alpha = jnp.sum(jnp.take(q, dst, axis=0) * jnp.take(k, src, axis=0),
                    axis=-1) * (1.0 / 16.0)
    a = jnp.exp(alpha)
    denom = jax.ops.segment_sum(a, dst, num_segments=N_NODES)
    acc = jax.ops.segment_sum(a[:, None] * jnp.take(v, src, axis=0), dst,
                              num_segments=N_NODES)
    acc2 = jnp.stack([acc[:, :128], acc[:, 128:]], axis=0)
    den_parts = denom[None, :]
    return acc2, den_parts


def kernel(geo_x, euc_x, edge_index, Wq, bq, Wk, bk, Wv, bv, Ws, bs, ln_w, ln_b):
    src = edge_index[0].astype(jnp.int32)
    dst = edge_index[1].astype(jnp.int32)
    q, skip, k, vab = _projections(euc_x, geo_x, Wq, bq, Wk, bk, Wv, bv, Ws, bs)
    acc, den_parts = _sparse_middle(q, k, vab, src, dst)
    return _finalize(acc, den_parts, skip, ln_w, ln_b)


# speedup vs baseline: 2.4786x; 1.5302x over previous
import functools

import jax
import jax.numpy as jnp
from jax import lax
from jax.experimental import pallas as pl
from jax.experimental.pallas import tpu as pltpu
from jax.experimental.pallas import tpu_sc as plsc

N_NODES = 10000
N_EDGES = 160000
DIM = 256

_NC = 2
_NS = 16
_NW = _NC * _NS
_EB = 32

_ACCW = 144


_ROWS = 1000


def _mm_body(eu_ref, geo_ref, wqs_ref, wkv_ref, bqs_ref, bkv_ref,
             q_ref, skip_ref, k_ref, vab_ref):
    qs = jnp.dot(eu_ref[...], wqs_ref[...],
                 preferred_element_type=jnp.float32) + bqs_ref[...]
    kv = jnp.dot(geo_ref[...], wkv_ref[...],
                 preferred_element_type=jnp.float32) + bkv_ref[...]
    q_ref[...] = qs[:, :DIM]
    skip_ref[...] = qs[:, DIM:]
    k_ref[...] = kv[:, :DIM]
    vab_ref[0] = kv[:, DIM:DIM + 128]
    vab_ref[1] = kv[:, DIM + 128:]


def _projections(euc_x, geo_x, Wq, bq, Wk, bk, Wv, bv, Ws, bs):
    wqs = jnp.concatenate([Wq, Ws], axis=1)
    wkv = jnp.concatenate([Wk, Wv], axis=1)
    bqs = jnp.concatenate([bq, bs])[None, :]
    bkv = jnp.concatenate([bk, bv])[None, :]
    grid = N_NODES // _ROWS
    return pl.pallas_call(
        _mm_body,
        grid=(grid,),
        in_specs=[
            pl.BlockSpec((_ROWS, DIM), lambda i: (i, 0)),
            pl.BlockSpec((_ROWS, DIM), lambda i: (i, 0)),
            pl.BlockSpec((DIM, 2 * DIM), lambda i: (0, 0)),
            pl.BlockSpec((DIM, 2 * DIM), lambda i: (0, 0)),
            pl.BlockSpec((1, 2 * DIM), lambda i: (0, 0)),
            pl.BlockSpec((1, 2 * DIM), lambda i: (0, 0)),
        ],
        out_specs=[
            pl.BlockSpec((_ROWS, DIM), lambda i: (i, 0)),
            pl.BlockSpec((_ROWS, DIM), lambda i: (i, 0)),
            pl.BlockSpec((_ROWS, DIM), lambda i: (i, 0)),
            pl.BlockSpec((2, _ROWS, 128), lambda i: (0, i, 0)),
        ],
        out_shape=[
            jax.ShapeDtypeStruct((N_NODES, DIM), jnp.float32),
            jax.ShapeDtypeStruct((N_NODES, DIM), jnp.float32),
            jax.ShapeDtypeStruct((N_NODES, DIM), jnp.float32),
            jax.ShapeDtypeStruct((2, N_NODES, 128), jnp.float32),
        ],
    )(euc_x, geo_x, wqs, wkv, bqs, bkv)


_A_FULL = 156
_A_TAIL_BASE = _NW * _A_FULL * _EB


def _edge_alpha(qbuf, kbuf, lanes):
    alphav = jnp.zeros((16,), jnp.float32)
    for e in range(16):
        acc = qbuf[e, pl.ds(0, 16)] * kbuf[e, pl.ds(0, 16)]
        for j in range(1, 16):
            acc = acc + qbuf[e, pl.ds(j * 16, 16)] * kbuf[e, pl.ds(j * 16, 16)]
        alpha_e = jnp.sum(acc) * 0.0625
        alphav = jnp.where(lanes == e, alpha_e, alphav)
    return jnp.exp(alphav)


def _alpha_body(q_hbm, k_hbm, src_hbm, dst_hbm, a_out,
                srcv, dstv, qbuf, kbuf, abuf, sem1, sem2):
    c = lax.axis_index("c")
    s = lax.axis_index("s")
    w = s * _NC + c
    lanes = lax.iota(jnp.int32, 16)

    def do_block(eb):
        pltpu.sync_copy(src_hbm.at[pl.ds(eb, _EB)], srcv)
        pltpu.sync_copy(dst_hbm.at[pl.ds(eb, _EB)], dstv)
        cp1 = pltpu.async_copy(q_hbm.at[dstv], qbuf, sem1)
        cp2 = pltpu.async_copy(k_hbm.at[srcv], kbuf, sem2)
        cp1.wait()
        cp2.wait()
        for h in range(2):
            av = _edge_alpha(qbuf.at[pl.ds(h * 16, 16)],
                             kbuf.at[pl.ds(h * 16, 16)], lanes)
            abuf[pl.ds(h * 16, 16)] = av
        pltpu.sync_copy(abuf, a_out.at[pl.ds(eb, _EB)])

    def loop_body(g, carry):
        do_block(w * _A_FULL * _EB + g * _EB)
        return carry

    lax.fori_loop(0, _A_FULL, loop_body, 0, unroll=False)

    @pl.when(w < 8)
    def _():
        do_block(_A_TAIL_BASE + w * _EB)


def _edge_coeffs(q, k, src, dst):
    mesh = plsc.VectorSubcoreMesh(core_axis_name="c", subcore_axis_name="s")
    f = pl.kernel(
        _alpha_body,
        out_type=jax.ShapeDtypeStruct((N_EDGES,), jnp.float32),
        mesh=mesh,
        compiler_params=pltpu.CompilerParams(needs_layout_passes=False, use_tc_tiling_on_sc=False),
        scratch_types=[
            pltpu.VMEM((_EB,), jnp.int32),
            pltpu.VMEM((_EB,), jnp.int32),
            pltpu.VMEM((_EB, DIM), jnp.float32),
            pltpu.VMEM((_EB, DIM), jnp.float32),
            pltpu.VMEM((_EB,), jnp.float32),
            pltpu.SemaphoreType.DMA,
            pltpu.SemaphoreType.DMA,
        ],
    )
    return f(q, k, src, dst)


_B_FULL = 312
_B_TAIL_BASE = _NS * _B_FULL * _EB
_NPAD = 10240
_RPT = _NPAD // _NS


def _agg_body(a_hbm, src_hbm, dst_hbm, vab_hbm, acc_out,
              srcv, dstv, idxv, avbuf, vbuf, wbuf, zbuf, acc_sh, sem1):
    c = lax.axis_index("c")
    s = lax.axis_index("s")
    lanes = lax.iota(jnp.int32, 16)
    unit = jnp.where(lanes == 0, 1.0, 0.0)

    zero = jnp.zeros((16,), jnp.float32)
    def zrow(i, carry):
        for j in range(_ACCW // 16):
            zbuf[i, pl.ds(j * 16, 16)] = zero
        return carry
    lax.fori_loop(0, 128, zrow, 0, unroll=False)
    for piece in range(_RPT // 128):
        pltpu.sync_copy(zbuf, acc_sh.at[pl.ds(s * _RPT + piece * 128, 128)])
    plsc.subcore_barrier()

    def do_block(eb):
        pltpu.sync_copy(src_hbm.at[pl.ds(eb, _EB)], srcv)
        pltpu.sync_copy(dst_hbm.at[pl.ds(eb, _EB)], dstv)
        pltpu.sync_copy(a_hbm.at[pl.ds(eb, _EB)], avbuf)
        base = c * N_NODES
        for h in range(2):
            idxv[pl.ds(h * 16, 16)] = srcv[pl.ds(h * 16, 16)] + base
        pltpu.async_copy(vab_hbm.at[idxv], vbuf, sem1).wait()
        for h in range(2):
            avchunk = avbuf[pl.ds(h * 16, 16)]
            for e16 in range(16):
                e = h * 16 + e16
                ae = avchunk[e16]
                for j in range(8):
                    wbuf[e, pl.ds(j * 16, 16)] = vbuf[e, pl.ds(j * 16, 16)] * ae
                wbuf[e, pl.ds(128, 16)] = unit * ae
        pltpu.sync_copy(wbuf, acc_sh.at[dstv], add=True)

    def loop_body(g, carry):
        do_block(s * _B_FULL * _EB + g * _EB)
        return carry

    lax.fori_loop(0, _B_FULL, loop_body, 0, unroll=False)

    @pl.when(s < 8)
    def _():
        do_block(_B_TAIL_BASE + s * _EB)

    plsc.subcore_barrier()
    pltpu.sync_copy(acc_sh.at[pl.ds(s * _RPT, _RPT)],
                    acc_out.at[c].at[pl.ds(s * _RPT, _RPT)])


def _aggregate(a, src, dst, vab2):
    mesh = plsc.VectorSubcoreMesh(core_axis_name="c", subcore_axis_name="s")
    f = pl.kernel(
        _agg_body,
        out_type=jax.ShapeDtypeStruct((_NC, _NPAD, _ACCW), jnp.float32),
        mesh=mesh,
        compiler_params=pltpu.CompilerParams(needs_layout_passes=False, use_tc_tiling_on_sc=False),
        scratch_types=[
            pltpu.VMEM((_EB,), jnp.int32),
            pltpu.VMEM((_EB,), jnp.int32),
            pltpu.VMEM((_EB,), jnp.int32),
            pltpu.VMEM((_EB,), jnp.float32),
            pltpu.VMEM((_EB, 128), jnp.float32),
            pltpu.VMEM((_EB, _ACCW), jnp.float32),
            pltpu.VMEM((128, _ACCW), jnp.float32),
            pltpu.VMEM_SHARED((_NPAD, _ACCW), jnp.float32),
            pltpu.SemaphoreType.DMA,
        ],
    )
    return f(a, src, dst, vab2)


def _finalize_body(acc_ref, skip_ref, lnw_ref, lnb_ref, out_ref):
    accA = acc_ref[0]
    accB = acc_ref[1]
    dA = accA[:, 128:129] + 1e-16
    dB = accB[:, 128:129] + 1e-16
    pre = jnp.concatenate([accA[:, :128] / dA, accB[:, :128] / dB], axis=1)
    pre = pre + skip_ref[...]
    mu = jnp.mean(pre)
    cen = pre - mu
    sig = jnp.sqrt(jnp.mean(cen * cen))
    out = cen / (sig + 1e-5) * lnw_ref[...] + lnb_ref[...]
    out_ref[...] = jnp.maximum(out, 0.0)


def _finalize(acc, skip, ln_w, ln_b):
    return pl.pallas_call(
        _finalize_body,
        out_shape=jax.ShapeDtypeStruct((N_NODES, DIM), jnp.float32),
    )(acc, skip, ln_w[None, :], ln_b[None, :])


def kernel(geo_x, euc_x, edge_index, Wq, bq, Wk, bk, Wv, bv, Ws, bs, ln_w, ln_b):
    src = edge_index[0].astype(jnp.int32)
    dst = edge_index[1].astype(jnp.int32)
    q, skip, k, vab = _projections(euc_x, geo_x, Wq, bq, Wk, bk, Wv, bv, Ws, bs)
    a = _edge_coeffs(q, k, src, dst)
    acc = _aggregate(a, src, dst, vab.reshape(2 * N_NODES, 128))
    return _finalize(acc[:, :N_NODES, :], skip, ln_w, ln_b)


# speedup vs baseline: 4.4778x; 1.8066x over previous
import functools

import jax
import jax.numpy as jnp
from jax import lax
from jax.experimental import pallas as pl
from jax.experimental.pallas import tpu as pltpu
from jax.experimental.pallas import tpu_sc as plsc

N_NODES = 10000
N_EDGES = 160000
DIM = 256

_NC = 2
_NS = 16
_NW = _NC * _NS
_EB = 64

_ACCW = 144
_NPAD = 10240

_SC_PARAMS = pltpu.CompilerParams(
    needs_layout_passes=False, use_tc_tiling_on_sc=False)


_ROWS = 1000


def _mm_body(eu_ref, geo_ref, wqs_ref, wkv_ref, bqs_ref, bkv_ref,
             q_ref, skip_ref, k_ref, vab_ref):
    qs = jnp.dot(eu_ref[...], wqs_ref[...],
                 preferred_element_type=jnp.float32) + bqs_ref[...]
    kv = jnp.dot(geo_ref[...], wkv_ref[...],
                 preferred_element_type=jnp.float32) + bkv_ref[...]
    q_ref[...] = qs[:, :DIM]
    skip_ref[...] = qs[:, DIM:]
    k_ref[...] = kv[:, :DIM]
    vab_ref[0] = kv[:, DIM:DIM + 128]
    vab_ref[1] = kv[:, DIM + 128:]


def _projections(euc_x, geo_x, Wq, bq, Wk, bk, Wv, bv, Ws, bs):
    wqs = jnp.concatenate([Wq, Ws], axis=1)
    wkv = jnp.concatenate([Wk, Wv], axis=1)
    bqs = jnp.concatenate([bq, bs])[None, :]
    bkv = jnp.concatenate([bk, bv])[None, :]
    grid = N_NODES // _ROWS
    return pl.pallas_call(
        _mm_body,
        grid=(grid,),
        in_specs=[
            pl.BlockSpec((_ROWS, DIM), lambda i: (i, 0)),
            pl.BlockSpec((_ROWS, DIM), lambda i: (i, 0)),
            pl.BlockSpec((DIM, 2 * DIM), lambda i: (0, 0)),
            pl.BlockSpec((DIM, 2 * DIM), lambda i: (0, 0)),
            pl.BlockSpec((1, 2 * DIM), lambda i: (0, 0)),
            pl.BlockSpec((1, 2 * DIM), lambda i: (0, 0)),
        ],
        out_specs=[
            pl.BlockSpec((_ROWS, DIM), lambda i: (i, 0)),
            pl.BlockSpec((_ROWS, DIM), lambda i: (i, 0)),
            pl.BlockSpec((_ROWS, DIM), lambda i: (i, 0)),
            pl.BlockSpec((2, _ROWS, 128), lambda i: (0, i, 0)),
        ],
        out_shape=[
            jax.ShapeDtypeStruct((N_NODES, DIM), jnp.float32),
            jax.ShapeDtypeStruct((N_NODES, DIM), jnp.float32),
            jax.ShapeDtypeStruct((N_NODES, DIM), jnp.float32),
            jax.ShapeDtypeStruct((2, N_NODES, 128), jnp.float32),
        ],
    )(euc_x, geo_x, wqs, wkv, bqs, bkv)


_A_FULL = 78
_A_TAIL_BASE = _NW * _A_FULL * _EB
_A_TAIL = (N_EDGES - _A_TAIL_BASE) // _EB


def _edge_alpha_group(qg, kg, lanes):
    alphav = jnp.zeros((16,), jnp.float32)
    for e in range(16):
        acc = qg[e, pl.ds(0, 16)] * kg[e, pl.ds(0, 16)]
        for j in range(1, 16):
            acc = acc + qg[e, pl.ds(j * 16, 16)] * kg[e, pl.ds(j * 16, 16)]
        alpha_e = jnp.sum(acc) * 0.0625
        alphav = jnp.where(lanes == e, alpha_e, alphav)
    return jnp.exp(alphav)


def _alpha_body(q_hbm, k_hbm, src_hbm, dst_hbm, a_out,
                srcv, dstv, qbuf, kbuf, abuf,
                si0, si1, sq0, sq1, sk0, sk1, sa0, sa1):
    c = lax.axis_index("c")
    s_ax = lax.axis_index("s")
    w = s_ax * _NC + c
    lanes = lax.iota(jnp.int32, 16)
    base = w * _A_FULL * _EB
    si = (si0, si1)
    sq = (sq0, sq1)
    sk = (sk0, sk1)
    sa = (sa0, sa1)

    def idx_issue(eb, slot):
        pltpu.async_copy(src_hbm.at[pl.ds(eb, _EB)], srcv.at[slot], si[slot])
        pltpu.async_copy(dst_hbm.at[pl.ds(eb, _EB)], dstv.at[slot], si[slot])

    def idx_wait(slot):
        pltpu.make_async_copy(
            src_hbm.at[pl.ds(0, _EB)], srcv.at[slot], si[slot]).wait()
        pltpu.make_async_copy(
            dst_hbm.at[pl.ds(0, _EB)], dstv.at[slot], si[slot]).wait()

    def rows_issue(slot):
        pltpu.async_copy(q_hbm.at[dstv.at[slot]], qbuf.at[slot], sq[slot])
        pltpu.async_copy(k_hbm.at[srcv.at[slot]], kbuf.at[slot], sk[slot])

    def rows_wait(slot):
        pltpu.make_async_copy(
            q_hbm.at[dstv.at[slot]], qbuf.at[slot], sq[slot]).wait()
        pltpu.make_async_copy(
            k_hbm.at[srcv.at[slot]], kbuf.at[slot], sk[slot]).wait()

    def out_wait(slot):
        pltpu.make_async_copy(
            abuf.at[slot], a_out.at[pl.ds(0, _EB)], sa[slot]).wait()

    def alpha_blk(slot):
        def grp_body(grp, carry):
            av = _edge_alpha_group(qbuf.at[slot].at[pl.ds(grp * 16, 16)],
                                   kbuf.at[slot].at[pl.ds(grp * 16, 16)],
                                   lanes)
            abuf[slot, pl.ds(grp * 16, 16)] = av
            return carry
        lax.fori_loop(0, _EB // 16, grp_body, 0, unroll=False)

    def step(g, slot, pred_next, pred_next2, pred_prev):
        @pl.when(pred_next)
        def _():
            idx_wait(1 - slot)
            rows_issue(1 - slot)
        rows_wait(slot)
        @pl.when(pred_prev)
        def _():
            out_wait(slot)
        alpha_blk(slot)
        pltpu.async_copy(abuf.at[slot], a_out.at[pl.ds(base + g * _EB, _EB)],
                         sa[slot])
        @pl.when(pred_next2)
        def _():
            idx_issue(base + (g + 2) * _EB, slot)

    idx_issue(base, 0)
    idx_issue(base + _EB, 1)
    idx_wait(0)
    rows_issue(0)

    _HALF = _A_FULL // 2

    def pair(i, carry):
        g0 = 2 * i
        step(g0, 0, i >= 0, i < _HALF - 1, i >= 1)
        step(g0 + 1, 1, i < _HALF - 1, i < _HALF - 1, i >= 1)
        return carry

    lax.fori_loop(0, _HALF, pair, 0, unroll=False)
    out_wait(0)
    out_wait(1)

    @pl.when(w < _A_TAIL)
    def _():
        eb = _A_TAIL_BASE + w * _EB
        pltpu.sync_copy(src_hbm.at[pl.ds(eb, _EB)], srcv.at[0])
        pltpu.sync_copy(dst_hbm.at[pl.ds(eb, _EB)], dstv.at[0])
        rows_issue(0)
        rows_wait(0)
        alpha_blk(0)
        pltpu.sync_copy(abuf.at[0], a_out.at[pl.ds(eb, _EB)])


def _edge_coeffs(q, k, src, dst):
    mesh = plsc.VectorSubcoreMesh(core_axis_name="c", subcore_axis_name="s")
    f = pl.kernel(
        _alpha_body,
        out_type=jax.ShapeDtypeStruct((N_EDGES,), jnp.float32),
        mesh=mesh,
        compiler_params=_SC_PARAMS,
        scratch_types=[
            pltpu.VMEM((2, _EB), jnp.int32),
            pltpu.VMEM((2, _EB), jnp.int32),
            pltpu.VMEM((2, _EB, DIM), jnp.float32),
            pltpu.VMEM((2, _EB, DIM), jnp.float32),
            pltpu.VMEM((2, _EB), jnp.float32),
        ] + [pltpu.SemaphoreType.DMA] * 8,
    )
    return f(q, k, src, dst)


_B_FULL = 156
_B_TAIL_BASE = _NS * _B_FULL * _EB
_B_TAIL = (N_EDGES - _B_TAIL_BASE) // _EB
_RPT = _NPAD // _NS


def _agg_body(a_hbm, src_hbm, dst_hbm, vab_hbm, acc_out,
              srcv, dstv, dsts, idxv, avbuf, vbuf, wbuf, acc_sh,
              si0, si1, sv0, sv1, ss0, ss1):
    c = lax.axis_index("c")
    s_ax = lax.axis_index("s")
    lanes = lax.iota(jnp.int32, 16)
    unit = jnp.where(lanes == 0, 1.0, 0.0)
    vbase = c * N_NODES
    ebase = s_ax * _B_FULL * _EB
    si = (si0, si1)
    sv = (sv0, sv1)
    ss = (ss0, ss1)

    zero = jnp.zeros((16,), jnp.float32)
    def zrow(i, carry):
        for j in range(_ACCW // 16):
            wbuf[0, i, pl.ds(j * 16, 16)] = zero
        return carry
    lax.fori_loop(0, _EB, zrow, 0, unroll=False)
    for piece in range(_RPT // _EB):
        pltpu.sync_copy(wbuf.at[0],
                        acc_sh.at[pl.ds(s_ax * _RPT + piece * _EB, _EB)])
    plsc.subcore_barrier()

    def idx_issue(eb, slot):
        pltpu.async_copy(src_hbm.at[pl.ds(eb, _EB)], srcv.at[slot], si[slot])
        pltpu.async_copy(dst_hbm.at[pl.ds(eb, _EB)], dstv.at[slot], si[slot])
        pltpu.async_copy(a_hbm.at[pl.ds(eb, _EB)], avbuf.at[slot], si[slot])

    def idx_wait(slot):
        pltpu.make_async_copy(
            src_hbm.at[pl.ds(0, _EB)], srcv.at[slot], si[slot]).wait()
        pltpu.make_async_copy(
            dst_hbm.at[pl.ds(0, _EB)], dstv.at[slot], si[slot]).wait()
        pltpu.make_async_copy(
            a_hbm.at[pl.ds(0, _EB)], avbuf.at[slot], si[slot]).wait()

    def rows_issue(slot):
        for j in range(_EB // 16):
            idxv[slot, pl.ds(j * 16, 16)] = (
                srcv[slot, pl.ds(j * 16, 16)] + vbase)
        pltpu.async_copy(vab_hbm.at[idxv.at[slot]], vbuf.at[slot], sv[slot])

    def rows_wait(slot):
        pltpu.make_async_copy(
            vab_hbm.at[idxv.at[slot]], vbuf.at[slot], sv[slot]).wait()

    def scat_wait(slot):
        pltpu.make_async_copy(
            wbuf.at[slot], acc_sh.at[dsts.at[slot]], ss[slot]).wait()

    def compute(slot):
        for j in range(_EB // 16):
            dsts[slot, pl.ds(j * 16, 16)] = dstv[slot, pl.ds(j * 16, 16)]
        def grp_body(grp, carry):
            av16 = avbuf[slot, pl.ds(grp * 16, 16)]
            vb = vbuf.at[slot].at[pl.ds(grp * 16, 16)]
            wb = wbuf.at[slot].at[pl.ds(grp * 16, 16)]
            for e in range(16):
                ae = av16[e]
                for j in range(8):
                    wb[e, pl.ds(j * 16, 16)] = vb[e, pl.ds(j * 16, 16)] * ae
                wb[e, pl.ds(128, 16)] = unit * ae
            return carry
        lax.fori_loop(0, _EB // 16, grp_body, 0, unroll=False)
        pltpu.async_copy(wbuf.at[slot], acc_sh.at[dsts.at[slot]], ss[slot],
                         add=True)

    def step(g, slot, pred_next, pred_next2, pred_prev):
        @pl.when(pred_next)
        def _():
            idx_wait(1 - slot)
            rows_issue(1 - slot)
        rows_wait(slot)
        @pl.when(pred_prev)
        def _():
            scat_wait(slot)
        compute(slot)
        @pl.when(pred_next2)
        def _():
            idx_issue(ebase + (g + 2) * _EB, slot)

    idx_issue(ebase, 0)
    idx_issue(ebase + _EB, 1)
    idx_wait(0)
    rows_issue(0)

    _HALF = _B_FULL // 2

    def pair(i, carry):
        g0 = 2 * i
        step(g0, 0, i >= 0, i < _HALF - 1, i >= 1)
        step(g0 + 1, 1, i < _HALF - 1, i < _HALF - 1, i >= 1)
        return carry

    lax.fori_loop(0, _HALF, pair, 0, unroll=False)
    scat_wait(0)
    scat_wait(1)

    @pl.when(s_ax < _B_TAIL)
    def _():
        eb = _B_TAIL_BASE + s_ax * _EB
        pltpu.sync_copy(src_hbm.at[pl.ds(eb, _EB)], srcv.at[0])
        pltpu.sync_copy(dst_hbm.at[pl.ds(eb, _EB)], dstv.at[0])
        pltpu.sync_copy(a_hbm.at[pl.ds(eb, _EB)], avbuf.at[0])
        rows_issue(0)
        rows_wait(0)
        compute(0)
        scat_wait(0)

    plsc.subcore_barrier()
    pltpu.sync_copy(acc_sh.at[pl.ds(s_ax * _RPT, _RPT)],
                    acc_out.at[c].at[pl.ds(s_ax * _RPT, _RPT)])


def _aggregate(a, src, dst, vab2):
    mesh = plsc.VectorSubcoreMesh(core_axis_name="c", subcore_axis_name="s")
    f = pl.kernel(
        _agg_body,
        out_type=jax.ShapeDtypeStruct((_NC, _NPAD, _ACCW), jnp.float32),
        mesh=mesh,
        compiler_params=_SC_PARAMS,
        scratch_types=[
            pltpu.VMEM((2, _EB), jnp.int32),
            pltpu.VMEM((2, _EB), jnp.int32),
            pltpu.VMEM((2, _EB), jnp.int32),
            pltpu.VMEM((2, _EB), jnp.int32),
            pltpu.VMEM((2, _EB), jnp.float32),
            pltpu.VMEM((2, _EB, 128), jnp.float32),
            pltpu.VMEM((2, _EB, _ACCW), jnp.float32),
            pltpu.VMEM_SHARED((_NPAD, _ACCW), jnp.float32),
        ] + [pltpu.SemaphoreType.DMA] * 6,
    )
    return f(a, src, dst, vab2)


def _finalize_body(acc_ref, skip_ref, lnw_ref, lnb_ref, out_ref):
    accA = acc_ref[0]
    accB = acc_ref[1]
    dA = accA[:, 128:129] + 1e-16
    dB = accB[:, 128:129] + 1e-16
    pre = jnp.concatenate([accA[:, :128] / dA, accB[:, :128] / dB], axis=1)
    pre = pre + skip_ref[...]
    mu = jnp.mean(pre)
    cen = pre - mu
    sig = jnp.sqrt(jnp.mean(cen * cen))
    out = cen / (sig + 1e-5) * lnw_ref[...] + lnb_ref[...]
    out_ref[...] = jnp.maximum(out, 0.0)


def _finalize(acc, skip, ln_w, ln_b):
    return pl.pallas_call(
        _finalize_body,
        out_shape=jax.ShapeDtypeStruct((N_NODES, DIM), jnp.float32),
    )(acc, skip, ln_w[None, :], ln_b[None, :])


def kernel(geo_x, euc_x, edge_index, Wq, bq, Wk, bk, Wv, bv, Ws, bs, ln_w, ln_b):
    src = edge_index[0].astype(jnp.int32)
    dst = edge_index[1].astype(jnp.int32)
    q, skip, k, vab = _projections(euc_x, geo_x, Wq, bq, Wk, bk, Wv, bv, Ws, bs)
    a = _edge_coeffs(q, k, src, dst)
    acc = _aggregate(a, src, dst, vab.reshape(2 * N_NODES, 128))
    return _finalize(acc[:, :N_NODES, :], skip, ln_w, ln_b)
